# Initial kernel scaffold; baseline (speedup 1.0000x reference)
import jax
import jax.numpy as jnp
from jax.experimental import pallas as pl


def kernel(features, Wq, Wv, Wo, W_off, b_off, W_att, b_att, gamma, beta):
    raise NotImplementedError("write your pallas kernel here")


# speedup vs baseline: 212.4418x; 212.4418x over previous
import numpy as np
import jax
import jax.numpy as jnp
from jax.experimental import pallas as pl

_B, _C, _D, _H, _W = 2, 384, 32, 16, 16
_HW = _H * _W
_S = _D * _HW
_HP = 6
_DC = 8
_ND = _D // _DC
_CC = 192
_NC = _C // _CC
_SC = 2048
_NS = _S // _SC


def _pe_dc() -> np.ndarray:
    pos = np.arange(_D, dtype=np.float32)[:, None]
    div = np.exp(np.arange(0, _C, 2, dtype=np.float32) * (-np.log(10000.0) / _C))
    pe = np.zeros((_D, _C), np.float32)
    pe[:, 0::2] = np.sin(pos * div)
    pe[:, 1::2] = np.cos(pos * div)
    return pe


def _qv_kernel(f_ref, pe_ref, wq_ref, wv_ref, v_ref, qp_ref):
    x = f_ref[0] + pe_ref[0].T[:, :, None]
    q = jax.lax.dot_general(wq_ref[...], x, (((1,), (0,)), ((), ())),
                            preferred_element_type=jnp.float32)
    v = jax.lax.dot_general(wv_ref[...], x, (((1,), (0,)), ((), ())),
                            preferred_element_type=jnp.float32)
    v_ref[0] = v
    qp_ref[0] = q.max(axis=2).T


def _mix_kernel(v_ref, qp_ref, woff_ref, boff_ref, watt_ref, batt_ref,
                wo_ref, out_ref, st_ref):
    ci = pl.program_id(1)
    qp = qp_ref[0]
    off = jax.lax.dot_general(qp, woff_ref[...], (((1,), (1,)), ((), ())),
                              preferred_element_type=jnp.float32) + boff_ref[...]
    att = jax.lax.dot_general(qp, watt_ref[...], (((1,), (1,)), ((), ())),
                              preferred_element_type=jnp.float32) + batt_ref[...]
    att = jax.nn.softmax(att, axis=-1)
    off = jnp.clip(off, 0.0, float(_D - 1))
    low = jnp.floor(off)
    frac = off - low
    lowi = low.astype(jnp.int32)
    upi = jnp.ceil(off).astype(jnp.int32)
    dio = jax.lax.broadcasted_iota(jnp.int32, (_D, _D), 1)
    m = jnp.zeros((_D, _D), jnp.float32)
    for p in range(_HP):
        a = att[:, p][:, None]
        fr = frac[:, p][:, None]
        l = lowi[:, p][:, None]
        u = upi[:, p][:, None]
        m = m + a * ((1.0 - fr) * (dio == l).astype(jnp.float32)
                     + fr * (dio == u).astype(jnp.float32))
    s = jax.lax.dot_general(m, v_ref[0], (((1,), (1,)), ((), ())),
                            preferred_element_type=jnp.float32)
    part = jax.lax.dot_general(wo_ref[0], s, (((1,), (1,)), ((), ())),
                               preferred_element_type=jnp.float32)

    @pl.when(ci == 0)
    def _():
        out_ref[0] = part

    @pl.when(ci != 0)
    def _():
        out_ref[0] = out_ref[0] + part

    @pl.when(ci == _NC - 1)
    def _():
        t = out_ref[0]
        st_ref[0, 0] = jnp.sum(t, axis=(1, 2))[:, None]
        st_ref[0, 1] = jnp.sum(t * t, axis=(1, 2))[:, None]


def _bn_kernel(op_ref, f_ref, st_ref, g_ref, b_ref, y_ref):
    n = float(_B * _S)
    ssum = st_ref[0, 0] + st_ref[1, 0]
    ssq = st_ref[0, 1] + st_ref[1, 1]
    mean = ssum / n
    var = ssq / n - mean * mean
    a = g_ref[...] * jax.lax.rsqrt(var + 1e-5)
    bb = b_ref[...] - mean * a
    y_ref[0] = a * op_ref[0] + bb + f_ref[0]


def kernel(features, Wq, Wv, Wo, W_off, b_off, W_att, b_att, gamma, beta):
    f4 = features.reshape(_B, _C, _D, _HW)
    pe_r = jnp.asarray(_pe_dc()).reshape(_ND, _DC, _C)

    v, qp = pl.pallas_call(
        _qv_kernel,
        grid=(_B, _ND),
        in_specs=[
            pl.BlockSpec((1, _C, _DC, _HW), lambda b, d: (b, 0, d, 0)),
            pl.BlockSpec((1, _DC, _C), lambda b, d: (d, 0, 0)),
            pl.BlockSpec((_C, _C), lambda b, d: (0, 0)),
            pl.BlockSpec((_C, _C), lambda b, d: (0, 0)),
        ],
        out_specs=[
            pl.BlockSpec((1, _C, _DC, _HW), lambda b, d: (b, 0, d, 0)),
            pl.BlockSpec((1, _DC, _C), lambda b, d: (b, d, 0)),
        ],
        out_shape=[
            jax.ShapeDtypeStruct((_B, _C, _D, _HW), jnp.float32),
            jax.ShapeDtypeStruct((_B, _D, _C), jnp.float32),
        ],
    )(f4, pe_r, Wq, Wv)

    wo_r = Wo.reshape(_C, _NC, _CC).transpose(1, 0, 2)
    boff = b_off.reshape(1, _HP)
    batt = b_att.reshape(1, _HP)

    out_pre, st = pl.pallas_call(
        _mix_kernel,
        grid=(_B, _NC),
        in_specs=[
            pl.BlockSpec((1, _CC, _D, _HW), lambda b, c: (b, c, 0, 0)),
            pl.BlockSpec((1, _D, _C), lambda b, c: (b, 0, 0)),
            pl.BlockSpec((_HP, _C), lambda b, c: (0, 0)),
            pl.BlockSpec((1, _HP), lambda b, c: (0, 0)),
            pl.BlockSpec((_HP, _C), lambda b, c: (0, 0)),
            pl.BlockSpec((1, _HP), lambda b, c: (0, 0)),
            pl.BlockSpec((1, _C, _CC), lambda b, c: (c, 0, 0)),
        ],
        out_specs=[
            pl.BlockSpec((1, _C, _D, _HW), lambda b, c: (b, 0, 0, 0)),
            pl.BlockSpec((1, 2, _C, 1), lambda b, c: (b, 0, 0, 0)),
        ],
        out_shape=[
            jax.ShapeDtypeStruct((_B, _C, _D, _HW), jnp.float32),
            jax.ShapeDtypeStruct((_B, 2, _C, 1), jnp.float32),
        ],
    )(v, qp, W_off, boff, W_att, batt, wo_r)

    f3 = features.reshape(_B, _C, _S)
    op3 = out_pre.reshape(_B, _C, _S)

    y = pl.pallas_call(
        _bn_kernel,
        grid=(_B, _NS),
        in_specs=[
            pl.BlockSpec((1, _C, _SC), lambda b, s: (b, 0, s)),
            pl.BlockSpec((1, _C, _SC), lambda b, s: (b, 0, s)),
            pl.BlockSpec((_B, 2, _C, 1), lambda b, s: (0, 0, 0, 0)),
            pl.BlockSpec((_C, 1), lambda b, s: (0, 0)),
            pl.BlockSpec((_C, 1), lambda b, s: (0, 0)),
        ],
        out_specs=pl.BlockSpec((1, _C, _SC), lambda b, s: (b, 0, s)),
        out_shape=jax.ShapeDtypeStruct((_B, _C, _S), jnp.float32),
    )(op3, f3, st, gamma.reshape(_C, 1), beta.reshape(_C, 1))

    return y.reshape(_B, _C, _D, _H, _W)


# speedup vs baseline: 216.1834x; 1.0176x over previous
import numpy as np
import jax
import jax.numpy as jnp
from jax.experimental import pallas as pl

_B, _C, _D, _H, _W = 2, 384, 32, 16, 16
_HW = _H * _W
_S = _D * _HW
_HP = 6
_DC = 8
_ND = _D // _DC
_CC = 192
_NC = _C // _CC
_SC = 2048
_NS = _S // _SC


def _pe_dc() -> np.ndarray:
    pos = np.arange(_D, dtype=np.float32)[:, None]
    div = np.exp(np.arange(0, _C, 2, dtype=np.float32) * (-np.log(10000.0) / _C))
    pe = np.zeros((_D, _C), np.float32)
    pe[:, 0::2] = np.sin(pos * div)
    pe[:, 1::2] = np.cos(pos * div)
    return pe


def _qv_kernel(f_ref, pe_ref, wq_ref, wv_ref, v_ref, qp_ref):
    x = (f_ref[0] + pe_ref[0].T[:, :, None]).astype(jnp.bfloat16)
    q = jax.lax.dot_general(wq_ref[...], x, (((1,), (0,)), ((), ())),
                            preferred_element_type=jnp.float32)
    v = jax.lax.dot_general(wv_ref[...], x, (((1,), (0,)), ((), ())),
                            preferred_element_type=jnp.float32)
    v_ref[0] = v.astype(jnp.bfloat16)
    qp_ref[0] = q.max(axis=2).T


def _mix_kernel(v_ref, qp_ref, woff_ref, boff_ref, watt_ref, batt_ref,
                wo_ref, out_ref, st_ref):
    ci = pl.program_id(1)
    qp = qp_ref[0]
    off = jax.lax.dot_general(qp, woff_ref[...], (((1,), (1,)), ((), ())),
                              preferred_element_type=jnp.float32) + boff_ref[...]
    att = jax.lax.dot_general(qp, watt_ref[...], (((1,), (1,)), ((), ())),
                              preferred_element_type=jnp.float32) + batt_ref[...]
    att = jax.nn.softmax(att, axis=-1)
    off = jnp.clip(off, 0.0, float(_D - 1))
    low = jnp.floor(off)
    frac = off - low
    lowi = low.astype(jnp.int32)
    upi = jnp.ceil(off).astype(jnp.int32)
    dio = jax.lax.broadcasted_iota(jnp.int32, (_D, _D), 1)
    m = jnp.zeros((_D, _D), jnp.float32)
    for p in range(_HP):
        a = att[:, p][:, None]
        fr = frac[:, p][:, None]
        l = lowi[:, p][:, None]
        u = upi[:, p][:, None]
        m = m + a * ((1.0 - fr) * (dio == l).astype(jnp.float32)
                     + fr * (dio == u).astype(jnp.float32))
    s = jax.lax.dot_general(m.astype(jnp.bfloat16), v_ref[0],
                            (((1,), (1,)), ((), ())),
                            preferred_element_type=jnp.float32)
    part = jax.lax.dot_general(wo_ref[0], s.astype(jnp.bfloat16),
                               (((1,), (1,)), ((), ())),
                               preferred_element_type=jnp.float32)

    @pl.when(ci == 0)
    def _():
        out_ref[0] = part.astype(jnp.bfloat16)

    @pl.when(ci != 0)
    def _():
        out_ref[0] = (out_ref[0].astype(jnp.float32) + part).astype(jnp.bfloat16)

    @pl.when(ci == _NC - 1)
    def _():
        t = out_ref[0].astype(jnp.float32)
        st_ref[0, 0] = jnp.sum(t, axis=(1, 2))[:, None]
        st_ref[0, 1] = jnp.sum(t * t, axis=(1, 2))[:, None]


def _bn_kernel(op_ref, f_ref, st_ref, g_ref, b_ref, y_ref):
    n = float(_B * _S)
    ssum = st_ref[0, 0] + st_ref[1, 0]
    ssq = st_ref[0, 1] + st_ref[1, 1]
    mean = ssum / n
    var = ssq / n - mean * mean
    a = g_ref[...] * jax.lax.rsqrt(var + 1e-5)
    bb = b_ref[...] - mean * a
    y_ref[0] = a * op_ref[0].astype(jnp.float32) + bb + f_ref[0]


def kernel(features, Wq, Wv, Wo, W_off, b_off, W_att, b_att, gamma, beta):
    f4 = features.reshape(_B, _C, _D, _HW)
    pe_r = jnp.asarray(_pe_dc()).reshape(_ND, _DC, _C)

    v, qp = pl.pallas_call(
        _qv_kernel,
        grid=(_B, _ND),
        in_specs=[
            pl.BlockSpec((1, _C, _DC, _HW), lambda b, d: (b, 0, d, 0)),
            pl.BlockSpec((1, _DC, _C), lambda b, d: (d, 0, 0)),
            pl.BlockSpec((_C, _C), lambda b, d: (0, 0)),
            pl.BlockSpec((_C, _C), lambda b, d: (0, 0)),
        ],
        out_specs=[
            pl.BlockSpec((1, _C, _DC, _HW), lambda b, d: (b, 0, d, 0)),
            pl.BlockSpec((1, _DC, _C), lambda b, d: (b, d, 0)),
        ],
        out_shape=[
            jax.ShapeDtypeStruct((_B, _C, _D, _HW), jnp.bfloat16),
            jax.ShapeDtypeStruct((_B, _D, _C), jnp.float32),
        ],
    )(f4, pe_r, Wq.astype(jnp.bfloat16), Wv.astype(jnp.bfloat16))

    wo_r = Wo.reshape(_C, _NC, _CC).transpose(1, 0, 2).astype(jnp.bfloat16)
    boff = b_off.reshape(1, _HP)
    batt = b_att.reshape(1, _HP)

    out_pre, st = pl.pallas_call(
        _mix_kernel,
        grid=(_B, _NC),
        in_specs=[
            pl.BlockSpec((1, _CC, _D, _HW), lambda b, c: (b, c, 0, 0)),
            pl.BlockSpec((1, _D, _C), lambda b, c: (b, 0, 0)),
            pl.BlockSpec((_HP, _C), lambda b, c: (0, 0)),
            pl.BlockSpec((1, _HP), lambda b, c: (0, 0)),
            pl.BlockSpec((_HP, _C), lambda b, c: (0, 0)),
            pl.BlockSpec((1, _HP), lambda b, c: (0, 0)),
            pl.BlockSpec((1, _C, _CC), lambda b, c: (c, 0, 0)),
        ],
        out_specs=[
            pl.BlockSpec((1, _C, _D, _HW), lambda b, c: (b, 0, 0, 0)),
            pl.BlockSpec((1, 2, _C, 1), lambda b, c: (b, 0, 0, 0)),
        ],
        out_shape=[
            jax.ShapeDtypeStruct((_B, _C, _D, _HW), jnp.bfloat16),
            jax.ShapeDtypeStruct((_B, 2, _C, 1), jnp.float32),
        ],
    )(v, qp, W_off, boff, W_att, batt, wo_r)

    f3 = features.reshape(_B, _C, _S)
    op3 = out_pre.reshape(_B, _C, _S)

    y = pl.pallas_call(
        _bn_kernel,
        grid=(_B, _NS),
        in_specs=[
            pl.BlockSpec((1, _C, _SC), lambda b, s: (b, 0, s)),
            pl.BlockSpec((1, _C, _SC), lambda b, s: (b, 0, s)),
            pl.BlockSpec((_B, 2, _C, 1), lambda b, s: (0, 0, 0, 0)),
            pl.BlockSpec((_C, 1), lambda b, s: (0, 0)),
            pl.BlockSpec((_C, 1), lambda b, s: (0, 0)),
        ],
        out_specs=pl.BlockSpec((1, _C, _SC), lambda b, s: (b, 0, s)),
        out_shape=jax.ShapeDtypeStruct((_B, _C, _S), jnp.float32),
    )(op3, f3, st, gamma.reshape(_C, 1), beta.reshape(_C, 1))

    return y.reshape(_B, _C, _D, _H, _W)
